# baseline (speedup 1.0000x reference)
import functools

import jax
import jax.numpy as jnp
from jax import lax
from jax.experimental import pallas as pl
from jax.experimental.pallas import tpu as pltpu
from jax.experimental.pallas import tpu_sc as plsc

N_NODES = 10000
IN_FEAT = 128
OUT_FEAT = 128

NPAD = 10240
NW = 32
ROWS_PER_TILE = NPAD // 16
CHUNK = 128


def _sc_body(feat_hbm, src_hbm, dst_hbm, seg_out, cnt_out,
             dst_all, src_v, d_cur, rows_v, ones_v, zc_v,
             seg_sh, cnt_sh, gsem,
             *, chunks_per_tile):
    cid = lax.axis_index("c")
    sid = lax.axis_index("s")

    cpt0, cpt1 = chunks_per_tile
    my_cpt = jnp.where(cid == 0, cpt0, cpt1)
    base_c = jnp.where(cid == 0, sid * cpt0, 16 * cpt0 + sid * cpt1)
    base_e = base_c * CHUNK

    @pl.when(cid == 0)
    def _():
        pltpu.sync_copy(dst_hbm.at[pl.ds(base_e, cpt0 * CHUNK)],
                        dst_all.at[pl.ds(0, cpt0 * CHUNK)])

    @pl.when(cid == 1)
    def _():
        pltpu.sync_copy(dst_hbm.at[pl.ds(base_e, cpt1 * CHUNK)],
                        dst_all.at[pl.ds(0, cpt1 * CHUNK)])

    zrow = jnp.zeros((16,), jnp.float32)

    def zero_rows(i, _):
        for j in range(IN_FEAT // 16):
            rows_v[i, pl.ds(j * 16, 16)] = zrow
        return 0
    lax.fori_loop(0, CHUNK, zero_rows, 0)

    def zero_zc(i, _):
        zc_v[pl.ds(i * 16, 16)] = zrow
        return 0
    lax.fori_loop(0, ROWS_PER_TILE // 16, zero_zc, 0)

    for j in range(CHUNK // 16):
        ones_v[pl.ds(j * 16, 16)] = jnp.ones((16,), jnp.float32)

    base_n = sid * ROWS_PER_TILE
    for t in range(ROWS_PER_TILE // CHUNK):
        pltpu.sync_copy(rows_v, seg_sh.at[pl.ds(base_n + t * CHUNK, CHUNK)])
    pltpu.sync_copy(zc_v, cnt_sh.at[pl.ds(base_n, ROWS_PER_TILE)])

    plsc.subcore_barrier()

    def edge_step(t, _):
        pltpu.sync_copy(src_hbm.at[pl.ds(base_e + t * CHUNK, CHUNK)], src_v)
        for j in range(CHUNK // 16):
            d_cur[pl.ds(j * 16, 16)] = dst_all[pl.ds(t * CHUNK + j * 16, 16)]
        pltpu.async_copy(feat_hbm.at[src_v], rows_v, gsem).wait()
        pltpu.sync_copy(rows_v, seg_sh.at[d_cur], add=True)
        pltpu.sync_copy(ones_v, cnt_sh.at[d_cur], add=True)
        return 0
    lax.fori_loop(0, my_cpt, edge_step, 0)

    plsc.subcore_barrier()

    pltpu.sync_copy(seg_sh.at[pl.ds(base_n, ROWS_PER_TILE)],
                    seg_out.at[cid, pl.ds(base_n, ROWS_PER_TILE)])
    pltpu.sync_copy(cnt_sh.at[pl.ds(base_n, ROWS_PER_TILE)],
                    cnt_out.at[cid, pl.ds(base_n, ROWS_PER_TILE)])


def _segment_sum_sc(features, src, dst, chunks_per_tile):
    mesh = plsc.VectorSubcoreMesh(core_axis_name="c", subcore_axis_name="s")
    body = functools.partial(_sc_body, chunks_per_tile=chunks_per_tile)
    cpt_max = max(chunks_per_tile)
    return pl.kernel(
        body,
        out_type=[
            jax.ShapeDtypeStruct((2, NPAD, IN_FEAT), jnp.float32),
            jax.ShapeDtypeStruct((2, NPAD), jnp.float32),
        ],
        mesh=mesh,
        scratch_types=[
            pltpu.VMEM((cpt_max * CHUNK,), jnp.int32),
            pltpu.VMEM((CHUNK,), jnp.int32),
            pltpu.VMEM((CHUNK,), jnp.int32),
            pltpu.VMEM((CHUNK, IN_FEAT), jnp.float32),
            pltpu.VMEM((CHUNK,), jnp.float32),
            pltpu.VMEM((ROWS_PER_TILE,), jnp.float32),
            pltpu.VMEM_SHARED((NPAD, IN_FEAT), jnp.float32),
            pltpu.VMEM_SHARED((NPAD,), jnp.float32),
            pltpu.SemaphoreType.DMA,
        ],
    )(features, src, dst)


def _tc_body(feat_ref, w_ref, seg_ref, cnt_ref, out_ref):
    i = pl.program_id(0)
    blk = feat_ref.shape[0]
    w = w_ref[...]
    nodes_rep = jnp.dot(feat_ref[...], w, preferred_element_type=jnp.float32)
    seg = seg_ref[0] + seg_ref[1]
    cnt = cnt_ref[0, pl.ds(i * blk, blk)] + cnt_ref[1, pl.ds(i * blk, blk)]
    agg = seg / jnp.maximum(cnt, 1.0)[:, None]
    msgs = jnp.dot(agg, w, preferred_element_type=jnp.float32)
    out_ref[:, :OUT_FEAT] = jnp.maximum(nodes_rep, 0.0)
    out_ref[:, OUT_FEAT:] = jnp.maximum(msgs, 0.0)


def _combine_tc(feat_pad, W, seg_p, cnt_p):
    blk = 1024
    grid = (NPAD // blk,)
    return pl.pallas_call(
        _tc_body,
        grid=grid,
        in_specs=[
            pl.BlockSpec((blk, IN_FEAT), lambda i: (i, 0)),
            pl.BlockSpec((IN_FEAT, OUT_FEAT), lambda i: (0, 0)),
            pl.BlockSpec((2, blk, IN_FEAT), lambda i: (0, i, 0)),
            pl.BlockSpec((2, NPAD), lambda i: (0, 0)),
        ],
        out_specs=pl.BlockSpec((blk, 2 * OUT_FEAT), lambda i: (i, 0)),
        out_shape=jax.ShapeDtypeStruct((NPAD, 2 * OUT_FEAT), jnp.float32),
    )(feat_pad, W, seg_p, cnt_p)


CORE0_SHARE = 0.1


def kernel(features, edge_index, W):
    n_edges = edge_index.shape[1]
    cpt = -(-n_edges // (NW * CHUNK))
    cpt += cpt % 2
    total_chunks = cpt * NW
    cpt0 = max(2, int(round(total_chunks * CORE0_SHARE / 16 / 2)) * 2)
    cpt1 = total_chunks // 16 - cpt0
    epad = total_chunks * CHUNK
    ei = edge_index.astype(jnp.int32)
    pad = epad - n_edges
    src = jnp.concatenate([ei[1], jnp.zeros((pad,), jnp.int32)])
    dst = jnp.concatenate([ei[0], jnp.full((pad,), N_NODES, jnp.int32)])

    seg_p, cnt_p = _segment_sum_sc(features, src, dst, (cpt0, cpt1))

    feat_pad = jnp.pad(features, ((0, NPAD - N_NODES), (0, 0)))
    out = _combine_tc(feat_pad, W, seg_p, cnt_p)
    return out[:N_NODES]


# speedup vs baseline: 1.2281x; 1.2281x over previous
import functools

import jax
import jax.numpy as jnp
from jax import lax
from jax.experimental import pallas as pl
from jax.experimental.pallas import tpu as pltpu
from jax.experimental.pallas import tpu_sc as plsc

N_NODES = 10000
IN_FEAT = 128
OUT_FEAT = 128

NPAD = 10240
NW = 32
ROWS_PER_TILE = NPAD // 16
CHUNK = 128


def _sc_body(feat_hbm, src_hbm, dst_hbm, seg_out, cnt_out,
             dst_all, src_v, d_cur, rows_v, ones_v, zc_v,
             seg_sh, cnt_sh, gsem,
             *, chunks_per_tile):
    cid = lax.axis_index("c")
    sid = lax.axis_index("s")

    cpt0, cpt1 = chunks_per_tile
    my_cpt = jnp.where(cid == 0, cpt0, cpt1)
    base_c = jnp.where(cid == 0, sid * cpt0, 16 * cpt0 + sid * cpt1)
    base_e = base_c * CHUNK

    @pl.when(cid == 0)
    def _():
        pltpu.sync_copy(dst_hbm.at[pl.ds(base_e, cpt0 * CHUNK)],
                        dst_all.at[pl.ds(0, cpt0 * CHUNK)])

    @pl.when(cid == 1)
    def _():
        pltpu.sync_copy(dst_hbm.at[pl.ds(base_e, cpt1 * CHUNK)],
                        dst_all.at[pl.ds(0, cpt1 * CHUNK)])

    zrow = jnp.zeros((16,), jnp.float32)

    def zero_rows(i, _):
        for j in range(IN_FEAT // 16):
            rows_v[i, pl.ds(j * 16, 16)] = zrow
        return 0
    lax.fori_loop(0, CHUNK, zero_rows, 0)

    def zero_zc(i, _):
        zc_v[pl.ds(i * 16, 16)] = zrow
        return 0
    lax.fori_loop(0, ROWS_PER_TILE // 16, zero_zc, 0)

    for j in range(CHUNK // 16):
        ones_v[pl.ds(j * 16, 16)] = jnp.ones((16,), jnp.float32)

    base_n = sid * ROWS_PER_TILE
    for t in range(ROWS_PER_TILE // CHUNK):
        pltpu.sync_copy(rows_v, seg_sh.at[pl.ds(base_n + t * CHUNK, CHUNK)])
    pltpu.sync_copy(zc_v, cnt_sh.at[pl.ds(base_n, ROWS_PER_TILE)])

    plsc.subcore_barrier()

    def edge_step(t, _):
        pltpu.sync_copy(src_hbm.at[pl.ds(base_e + t * CHUNK, CHUNK)], src_v)
        for j in range(CHUNK // 16):
            d_cur[pl.ds(j * 16, 16)] = dst_all[pl.ds(t * CHUNK + j * 16, 16)]
        pltpu.async_copy(feat_hbm.at[src_v], rows_v, gsem).wait()
        pltpu.sync_copy(rows_v, seg_sh.at[d_cur], add=True)
        pltpu.sync_copy(ones_v, cnt_sh.at[d_cur], add=True)
        return 0
    lax.fori_loop(0, my_cpt, edge_step, 0)

    plsc.subcore_barrier()

    pltpu.sync_copy(seg_sh.at[pl.ds(base_n, ROWS_PER_TILE)],
                    seg_out.at[cid, pl.ds(base_n, ROWS_PER_TILE)])
    pltpu.sync_copy(cnt_sh.at[pl.ds(base_n, ROWS_PER_TILE)],
                    cnt_out.at[cid, pl.ds(base_n, ROWS_PER_TILE)])


def _segment_sum_sc(features, src, dst, chunks_per_tile):
    mesh = plsc.VectorSubcoreMesh(core_axis_name="c", subcore_axis_name="s")
    body = functools.partial(_sc_body, chunks_per_tile=chunks_per_tile)
    cpt_max = max(chunks_per_tile)
    return pl.kernel(
        body,
        out_type=[
            jax.ShapeDtypeStruct((2, NPAD, IN_FEAT), jnp.float32),
            jax.ShapeDtypeStruct((2, NPAD), jnp.float32),
        ],
        mesh=mesh,
        scratch_types=[
            pltpu.VMEM((cpt_max * CHUNK,), jnp.int32),
            pltpu.VMEM((CHUNK,), jnp.int32),
            pltpu.VMEM((CHUNK,), jnp.int32),
            pltpu.VMEM((CHUNK, IN_FEAT), jnp.float32),
            pltpu.VMEM((CHUNK,), jnp.float32),
            pltpu.VMEM((ROWS_PER_TILE,), jnp.float32),
            pltpu.VMEM_SHARED((NPAD, IN_FEAT), jnp.float32),
            pltpu.VMEM_SHARED((NPAD,), jnp.float32),
            pltpu.SemaphoreType.DMA,
        ],
    )(features, src, dst)


def _tc_body(feat_ref, w_ref, seg_ref, cnt_ref, out_ref):
    i = pl.program_id(0)
    blk = feat_ref.shape[0]
    w = w_ref[...]
    nodes_rep = jnp.dot(feat_ref[...], w, preferred_element_type=jnp.float32)
    seg = seg_ref[0] + seg_ref[1]
    cnt = cnt_ref[0, pl.ds(i * blk, blk)] + cnt_ref[1, pl.ds(i * blk, blk)]
    agg = seg / jnp.maximum(cnt, 1.0)[:, None]
    msgs = jnp.dot(agg, w, preferred_element_type=jnp.float32)
    out_ref[:, :OUT_FEAT] = jnp.maximum(nodes_rep, 0.0)
    out_ref[:, OUT_FEAT:] = jnp.maximum(msgs, 0.0)


def _combine_tc(feat_pad, W, seg_p, cnt_p):
    blk = 1024
    grid = (NPAD // blk,)
    return pl.pallas_call(
        _tc_body,
        grid=grid,
        in_specs=[
            pl.BlockSpec((blk, IN_FEAT), lambda i: (i, 0)),
            pl.BlockSpec((IN_FEAT, OUT_FEAT), lambda i: (0, 0)),
            pl.BlockSpec((2, blk, IN_FEAT), lambda i: (0, i, 0)),
            pl.BlockSpec((2, NPAD), lambda i: (0, 0)),
        ],
        out_specs=pl.BlockSpec((blk, 2 * OUT_FEAT), lambda i: (i, 0)),
        out_shape=jax.ShapeDtypeStruct((NPAD, 2 * OUT_FEAT), jnp.float32),
    )(feat_pad, W, seg_p, cnt_p)


CORE0_SHARE = 0.5


def kernel(features, edge_index, W):
    n_edges = edge_index.shape[1]
    cpt = -(-n_edges // (NW * CHUNK))
    cpt += cpt % 2
    total_chunks = cpt * NW
    cpt0 = max(2, int(round(total_chunks * CORE0_SHARE / 16 / 2)) * 2)
    cpt1 = total_chunks // 16 - cpt0
    epad = total_chunks * CHUNK
    ei = edge_index.astype(jnp.int32)
    pad = epad - n_edges
    src = jnp.concatenate([ei[1], jnp.zeros((pad,), jnp.int32)])
    dst = jnp.concatenate([ei[0], jnp.full((pad,), N_NODES, jnp.int32)])

    seg_p, cnt_p = _segment_sum_sc(features, src, dst, (cpt0, cpt1))

    feat_pad = jnp.pad(features, ((0, NPAD - N_NODES), (0, 0)))
    out = _combine_tc(feat_pad, W, seg_p, cnt_p)
    return out[:N_NODES]


# speedup vs baseline: 1.2288x; 1.0006x over previous
import functools

import jax
import jax.numpy as jnp
from jax import lax
from jax.experimental import pallas as pl
from jax.experimental.pallas import tpu as pltpu
from jax.experimental.pallas import tpu_sc as plsc

N_NODES = 10000
IN_FEAT = 128
OUT_FEAT = 128

NPAD = 10240
NW = 32
ROWS_PER_TILE = NPAD // 16
CHUNK = 128


def _sc_body(feat_hbm, src_hbm, dst_hbm, seg_out, cnt_out,
             dst_all, src_v, d_cur, rows_v, ones_v, zc_v,
             seg_sh, cnt_sh, gsem,
             *, chunks_per_tile):
    cid = lax.axis_index("c")
    sid = lax.axis_index("s")

    cpt0, cpt1 = chunks_per_tile
    my_cpt = jnp.where(cid == 0, cpt0, cpt1)
    base_c = jnp.where(cid == 0, sid * cpt0, 16 * cpt0 + sid * cpt1)
    base_e = base_c * CHUNK

    @pl.when(cid == 0)
    def _():
        pltpu.sync_copy(dst_hbm.at[pl.ds(base_e, cpt0 * CHUNK)],
                        dst_all.at[pl.ds(0, cpt0 * CHUNK)])

    @pl.when(cid == 1)
    def _():
        pltpu.sync_copy(dst_hbm.at[pl.ds(base_e, cpt1 * CHUNK)],
                        dst_all.at[pl.ds(0, cpt1 * CHUNK)])

    zrow = jnp.zeros((16,), jnp.float32)

    def zero_rows(i, _):
        for j in range(IN_FEAT // 16):
            rows_v[i, pl.ds(j * 16, 16)] = zrow
        return 0
    lax.fori_loop(0, CHUNK, zero_rows, 0)

    def zero_zc(i, _):
        zc_v[pl.ds(i * 16, 16)] = zrow
        return 0
    lax.fori_loop(0, ROWS_PER_TILE // 16, zero_zc, 0)

    for j in range(CHUNK // 16):
        ones_v[pl.ds(j * 16, 16)] = jnp.ones((16,), jnp.float32)

    base_n = sid * ROWS_PER_TILE
    for t in range(ROWS_PER_TILE // CHUNK):
        pltpu.sync_copy(rows_v, seg_sh.at[pl.ds(base_n + t * CHUNK, CHUNK)])
    pltpu.sync_copy(zc_v, cnt_sh.at[pl.ds(base_n, ROWS_PER_TILE)])

    plsc.subcore_barrier()

    def edge_step(t, _):
        pltpu.sync_copy(src_hbm.at[pl.ds(base_e + t * CHUNK, CHUNK)], src_v)
        for j in range(CHUNK // 16):
            d_cur[pl.ds(j * 16, 16)] = dst_all[pl.ds(t * CHUNK + j * 16, 16)]
        pltpu.async_copy(feat_hbm.at[src_v], rows_v, gsem).wait()
        pltpu.sync_copy(rows_v, seg_sh.at[d_cur], add=True)
        pltpu.sync_copy(ones_v, cnt_sh.at[d_cur], add=True)
        return 0

    @pl.when(cid == 0)
    def _():
        lax.fori_loop(0, cpt0, edge_step, 0)

    @pl.when(cid == 1)
    def _():
        lax.fori_loop(0, cpt1, edge_step, 0)

    plsc.subcore_barrier()

    pltpu.sync_copy(seg_sh.at[pl.ds(base_n, ROWS_PER_TILE)],
                    seg_out.at[cid, pl.ds(base_n, ROWS_PER_TILE)])
    pltpu.sync_copy(cnt_sh.at[pl.ds(base_n, ROWS_PER_TILE)],
                    cnt_out.at[cid, pl.ds(base_n, ROWS_PER_TILE)])


def _segment_sum_sc(features, src, dst, chunks_per_tile):
    mesh = plsc.VectorSubcoreMesh(core_axis_name="c", subcore_axis_name="s")
    body = functools.partial(_sc_body, chunks_per_tile=chunks_per_tile)
    cpt_max = max(chunks_per_tile)
    return pl.kernel(
        body,
        out_type=[
            jax.ShapeDtypeStruct((2, NPAD, IN_FEAT), jnp.float32),
            jax.ShapeDtypeStruct((2, NPAD), jnp.float32),
        ],
        mesh=mesh,
        scratch_types=[
            pltpu.VMEM((cpt_max * CHUNK,), jnp.int32),
            pltpu.VMEM((CHUNK,), jnp.int32),
            pltpu.VMEM((CHUNK,), jnp.int32),
            pltpu.VMEM((CHUNK, IN_FEAT), jnp.float32),
            pltpu.VMEM((CHUNK,), jnp.float32),
            pltpu.VMEM((ROWS_PER_TILE,), jnp.float32),
            pltpu.VMEM_SHARED((NPAD, IN_FEAT), jnp.float32),
            pltpu.VMEM_SHARED((NPAD,), jnp.float32),
            pltpu.SemaphoreType.DMA,
        ],
    )(features, src, dst)


def _tc_body(feat_ref, w_ref, seg_ref, cnt_ref, out_ref):
    i = pl.program_id(0)
    blk = feat_ref.shape[0]
    w = w_ref[...]
    nodes_rep = jnp.dot(feat_ref[...], w, preferred_element_type=jnp.float32)
    seg = seg_ref[0] + seg_ref[1]
    cnt = cnt_ref[0, pl.ds(i * blk, blk)] + cnt_ref[1, pl.ds(i * blk, blk)]
    agg = seg / jnp.maximum(cnt, 1.0)[:, None]
    msgs = jnp.dot(agg, w, preferred_element_type=jnp.float32)
    out_ref[:, :OUT_FEAT] = jnp.maximum(nodes_rep, 0.0)
    out_ref[:, OUT_FEAT:] = jnp.maximum(msgs, 0.0)


def _combine_tc(feat_pad, W, seg_p, cnt_p):
    blk = 1024
    grid = (NPAD // blk,)
    return pl.pallas_call(
        _tc_body,
        grid=grid,
        in_specs=[
            pl.BlockSpec((blk, IN_FEAT), lambda i: (i, 0)),
            pl.BlockSpec((IN_FEAT, OUT_FEAT), lambda i: (0, 0)),
            pl.BlockSpec((2, blk, IN_FEAT), lambda i: (0, i, 0)),
            pl.BlockSpec((2, NPAD), lambda i: (0, 0)),
        ],
        out_specs=pl.BlockSpec((blk, 2 * OUT_FEAT), lambda i: (i, 0)),
        out_shape=jax.ShapeDtypeStruct((NPAD, 2 * OUT_FEAT), jnp.float32),
    )(feat_pad, W, seg_p, cnt_p)


CORE0_SHARE = 0.5


def kernel(features, edge_index, W):
    n_edges = edge_index.shape[1]
    cpt = -(-n_edges // (NW * CHUNK))
    cpt += cpt % 2
    total_chunks = cpt * NW
    cpt0 = max(2, int(round(total_chunks * CORE0_SHARE / 16 / 2)) * 2)
    cpt1 = total_chunks // 16 - cpt0
    epad = total_chunks * CHUNK
    ei = edge_index.astype(jnp.int32)
    pad = epad - n_edges
    src = jnp.concatenate([ei[1], jnp.zeros((pad,), jnp.int32)])
    dst = jnp.concatenate([ei[0], jnp.full((pad,), N_NODES, jnp.int32)])

    seg_p, cnt_p = _segment_sum_sc(features, src, dst, (cpt0, cpt1))

    feat_pad = jnp.pad(features, ((0, NPAD - N_NODES), (0, 0)))
    out = _combine_tc(feat_pad, W, seg_p, cnt_p)
    return out[:N_NODES]
